# baseline (device time: 105471 ns/iter reference)
import jax
import jax.numpy as jnp
import numpy as np
from jax import lax
from jax.experimental import pallas as pl
from jax.experimental.pallas import tpu as pltpu

N_DEV = 32
NF = 16
NB = 15
NSUB = 4


def _ring_tables():
    plane = [(0, 0), (1, 0), (1, 1), (0, 1), (0, 2), (1, 2), (1, 3), (0, 3)]
    coord_to_mesh = {
        (x, y, z): z * 8 + i for z in range(4) for i, (x, y) in enumerate(plane)
    }
    yz = []
    for y in range(4):
        zs = range(4) if y % 2 == 0 else range(3, -1, -1)
        yz.extend((y, z) for z in zs)
    cycle = [(0, y, z) for (y, z) in yz] + [(1, y, z) for (y, z) in reversed(yz)]
    ring = [coord_to_mesh[c] for c in cycle]
    pos = [0] * N_DEV
    for r, m in enumerate(ring):
        pos[m] = r
    return np.array(ring, np.int32), np.array(pos, np.int32)


_RING, _POS = _ring_tables()


def kernel(x, w_mat, scale_x, scale_w):
    m_per, k = x.shape
    _, n_per = w_mat.shape

    def body(x_ref, w_ref, sx_ref, sw_ref, ring_ref, pos_ref, out_ref,
             comm_ref, fsend, frecv, bsend, brecv):
        my = lax.axis_index("i")
        r = pos_ref[my]
        left = ring_ref[lax.rem(r - 1 + N_DEV, N_DEV)]
        right = ring_ref[lax.rem(r + 1, N_DEV)]

        barrier = pltpu.get_barrier_semaphore()
        for nbr in (left, right):
            pl.semaphore_signal(
                barrier, inc=1,
                device_id=(nbr,), device_id_type=pl.DeviceIdType.MESH,
            )
        pl.semaphore_wait(barrier, 2)

        comm_ref[0] = x_ref[...].astype(jnp.float8_e5m2)

        m_sub = m_per // NSUB

        def fwd_desc(h, s):
            rows = pl.ds(s * m_sub, m_sub)
            return pltpu.make_async_remote_copy(
                src_ref=comm_ref.at[h - 1, rows, :],
                dst_ref=comm_ref.at[h, rows, :],
                send_sem=fsend.at[h - 1, s],
                recv_sem=frecv.at[h - 1, s],
                device_id=(right,),
                device_id_type=pl.DeviceIdType.MESH,
            )

        def bwd_desc(h, s):
            src = 0 if h == 1 else NF + h - 1
            rows = pl.ds(s * m_sub, m_sub)
            return pltpu.make_async_remote_copy(
                src_ref=comm_ref.at[src, rows, :],
                dst_ref=comm_ref.at[NF + h, rows, :],
                send_sem=bsend.at[h - 1, s],
                recv_sem=brecv.at[h - 1, s],
                device_id=(left,),
                device_id_type=pl.DeviceIdType.MESH,
            )

        fwd = {(1, s): fwd_desc(1, s) for s in range(NSUB)}
        bwd = {(1, s): bwd_desc(1, s) for s in range(NSUB)}
        for s in range(NSUB):
            fwd[1, s].start()
            bwd[1, s].start()

        scale = sx_ref[0] * sw_ref[0]
        w_bf = w_ref[...].astype(jnp.bfloat16)

        def block(chunk_slot, origin):
            out_ref[pl.ds(origin * m_per, m_per), :] = (
                jnp.dot(comm_ref[chunk_slot].astype(jnp.bfloat16), w_bf,
                        preferred_element_type=jnp.float32) * scale
            )

        block(0, my)

        for h in range(1, NF + 1):
            for s in range(NSUB):
                fwd[h, s].wait_recv()
                if h + 1 <= NF:
                    fwd[h + 1, s] = fwd_desc(h + 1, s)
                    fwd[h + 1, s].start()
                if h <= NB:
                    bwd[h, s].wait_recv()
                    if h + 1 <= NB:
                        bwd[h + 1, s] = bwd_desc(h + 1, s)
                        bwd[h + 1, s].start()
            block(h, ring_ref[lax.rem(r - h + N_DEV, N_DEV)])
            if h <= NB:
                block(NF + h, ring_ref[lax.rem(r + h, N_DEV)])

        for h in range(1, NF + 1):
            for s in range(NSUB):
                fwd[h, s].wait_send()
        for h in range(1, NB + 1):
            for s in range(NSUB):
                bwd[h, s].wait_send()

    return pl.pallas_call(
        body,
        out_shape=jax.ShapeDtypeStruct((N_DEV * m_per, n_per), jnp.float32),
        in_specs=[
            pl.BlockSpec(memory_space=pltpu.VMEM),
            pl.BlockSpec(memory_space=pltpu.VMEM),
            pl.BlockSpec(memory_space=pltpu.SMEM),
            pl.BlockSpec(memory_space=pltpu.SMEM),
            pl.BlockSpec(memory_space=pltpu.SMEM),
            pl.BlockSpec(memory_space=pltpu.SMEM),
        ],
        out_specs=pl.BlockSpec(memory_space=pltpu.VMEM),
        scratch_shapes=[
            pltpu.VMEM((N_DEV, m_per, k), jnp.float8_e5m2),
            pltpu.SemaphoreType.DMA((NF, NSUB)),
            pltpu.SemaphoreType.DMA((NF, NSUB)),
            pltpu.SemaphoreType.DMA((NB, NSUB)),
            pltpu.SemaphoreType.DMA((NB, NSUB)),
        ],
        compiler_params=pltpu.CompilerParams(collective_id=0),
    )(x, w_mat, scale_x, scale_w,
      jnp.asarray(_RING), jnp.asarray(_POS))


# device time: 103936 ns/iter; 1.0148x vs baseline; 1.0148x over previous
import jax
import jax.numpy as jnp
import numpy as np
from jax import lax
from jax.experimental import pallas as pl
from jax.experimental.pallas import tpu as pltpu

N_DEV = 32
NF = 16
NSUB = 2


def _ring_tables():
    plane = [(0, 0), (1, 0), (1, 1), (0, 1), (0, 2), (1, 2), (1, 3), (0, 3)]
    coord_to_mesh = {
        (x, y, z): z * 8 + i for z in range(4) for i, (x, y) in enumerate(plane)
    }
    yz = []
    for y in range(4):
        zs = range(4) if y % 2 == 0 else range(3, -1, -1)
        yz.extend((y, z) for z in zs)
    cycle = [(0, y, z) for (y, z) in yz] + [(1, y, z) for (y, z) in reversed(yz)]
    ring = [coord_to_mesh[c] for c in cycle]
    pos = [0] * N_DEV
    for r, m in enumerate(ring):
        pos[m] = r
    return np.array(ring, np.int32), np.array(pos, np.int32)


_RING, _POS = _ring_tables()


def kernel(x, w_mat, scale_x, scale_w):
    m_per, k = x.shape
    _, n_per = w_mat.shape

    def body(x_ref, w_ref, sx_ref, sw_ref, ring_ref, pos_ref, out_ref,
             comm_ref, fsend, frecv, bsend, brecv):
        my = lax.axis_index("i")
        r = pos_ref[my]
        left = ring_ref[lax.rem(r - 1 + N_DEV, N_DEV)]
        right = ring_ref[lax.rem(r + 1, N_DEV)]

        barrier = pltpu.get_barrier_semaphore()
        for nbr in (left, right):
            pl.semaphore_signal(
                barrier, inc=1,
                device_id=(nbr,), device_id_type=pl.DeviceIdType.MESH,
            )
        pl.semaphore_wait(barrier, 2)

        comm_ref[0] = x_ref[...].astype(jnp.float8_e5m2)

        m_sub = m_per // NSUB

        def fwd_desc(h, s):
            rows = pl.ds(s * m_sub, m_sub)
            return pltpu.make_async_remote_copy(
                src_ref=comm_ref.at[h - 1, rows, :],
                dst_ref=comm_ref.at[h, rows, :],
                send_sem=fsend.at[h - 1, s],
                recv_sem=frecv.at[h - 1, s],
                device_id=(right,),
                device_id_type=pl.DeviceIdType.MESH,
            )

        def bwd_desc(h, s):
            src = 0 if h == 1 else NF + h - 1
            dst = NF if h == NF else NF + h
            rows = pl.ds(s * m_sub, m_sub)
            return pltpu.make_async_remote_copy(
                src_ref=comm_ref.at[src, rows, :],
                dst_ref=comm_ref.at[dst, rows, :],
                send_sem=bsend.at[h - 1, s],
                recv_sem=brecv.at[h - 1, s],
                device_id=(left,),
                device_id_type=pl.DeviceIdType.MESH,
            )

        fwd = {(1, s): fwd_desc(1, s) for s in range(NSUB)}
        bwd = {(1, s): bwd_desc(1, s) for s in range(NSUB)}
        for s in range(NSUB):
            fwd[1, s].start()
            bwd[1, s].start()

        scale = sx_ref[0] * sw_ref[0]
        w_bf = w_ref[...].astype(jnp.bfloat16)

        def block(chunk_slot, origin):
            out_ref[pl.ds(origin * m_per, m_per), :] = (
                jnp.dot(comm_ref[chunk_slot].astype(jnp.bfloat16), w_bf,
                        preferred_element_type=jnp.float32) * scale
            )

        block(0, my)

        for h in range(1, NF):
            for s in range(NSUB):
                fwd[h, s].wait_recv()
                if h + 1 < NF or s < NSUB // 2:
                    fwd[h + 1, s] = fwd_desc(h + 1, s)
                    fwd[h + 1, s].start()
                bwd[h, s].wait_recv()
                if h + 1 < NF or s >= NSUB // 2:
                    bwd[h + 1, s] = bwd_desc(h + 1, s)
                    bwd[h + 1, s].start()
            block(h, ring_ref[lax.rem(r - h + N_DEV, N_DEV)])
            block(NF + h, ring_ref[lax.rem(r + h, N_DEV)])

        for s in range(NSUB // 2):
            fwd[NF, s].wait_recv()
        for s in range(NSUB // 2, NSUB):
            bwd[NF, s].wait_recv()
        block(NF, ring_ref[lax.rem(r + NF, N_DEV)])

        for d in fwd.values():
            d.wait_send()
        for d in bwd.values():
            d.wait_send()

    return pl.pallas_call(
        body,
        out_shape=jax.ShapeDtypeStruct((N_DEV * m_per, n_per), jnp.float32),
        in_specs=[
            pl.BlockSpec(memory_space=pltpu.VMEM),
            pl.BlockSpec(memory_space=pltpu.VMEM),
            pl.BlockSpec(memory_space=pltpu.SMEM),
            pl.BlockSpec(memory_space=pltpu.SMEM),
            pl.BlockSpec(memory_space=pltpu.SMEM),
            pl.BlockSpec(memory_space=pltpu.SMEM),
        ],
        out_specs=pl.BlockSpec(memory_space=pltpu.VMEM),
        scratch_shapes=[
            pltpu.VMEM((N_DEV, m_per, k), jnp.float8_e5m2),
            pltpu.SemaphoreType.DMA((NF, NSUB)),
            pltpu.SemaphoreType.DMA((NF, NSUB)),
            pltpu.SemaphoreType.DMA((NF, NSUB)),
            pltpu.SemaphoreType.DMA((NF, NSUB)),
        ],
        compiler_params=pltpu.CompilerParams(collective_id=0),
    )(x, w_mat, scale_x, scale_w,
      jnp.asarray(_RING), jnp.asarray(_POS))
